# baseline (speedup 1.0000x reference)
import functools

import jax
import jax.numpy as jnp
from jax import lax
from jax.experimental import pallas as pl
from jax.experimental.pallas import tpu as pltpu
from jax.experimental.pallas import tpu_sc as plsc

N_ROWS = 1000000
EMBED_DIM = 16
BATCH = 16384
HIST = 50

NC = 2
NS = 16
NW = NC * NS
TJ = BATCH // 128
TJ_PER_W = TJ // NW
BW = 128 * TJ_PER_W


def _body(
    xT_hbm, w_hbm, out_hbm,
    idx_all, rows_a, rows_b, tbuf_a, tbuf_b,
    sem_g, sem_o,
):
    wid = lax.axis_index("s") * NC + lax.axis_index("c")
    base = wid * BW

    d_iota = lax.iota(jnp.int32, EMBED_DIM)
    ti_idx = d_iota // 8
    r_idx = d_iota % 8
    zeros = jnp.zeros((EMBED_DIM,), jnp.int32)

    pltpu.sync_copy(xT_hbm.at[:, pl.ds(base, BW)], idx_all)

    def gather(h, rows_v):
        return pltpu.async_copy(w_hbm.at[idx_all.at[h]], rows_v, sem_g)

    def transpose(rows_v, tbuf):
        for j in range(TJ_PER_W):
            j_splat = zeros + j

            def per_c(c, carry):
                vals = rows_v[j * 128 + c]
                plsc.store_scatter(
                    tbuf, [ti_idx, j_splat, r_idx, zeros + c], vals
                )
                return carry

            lax.fori_loop(0, 128, per_c, 0, unroll=16)

    def write_out(h, tbuf):
        pltpu.async_copy(
            tbuf, out_hbm.at[h, :, pl.ds(wid * TJ_PER_W, TJ_PER_W)], sem_o
        )

    def drain_write(h, tbuf):
        pltpu.make_async_copy(
            tbuf, out_hbm.at[h, :, pl.ds(wid * TJ_PER_W, TJ_PER_W)], sem_o
        ).wait()

    gather(0, rows_a)

    def step(k, carry):
        h0 = 2 * k
        h1 = 2 * k + 1
        gather(h1, rows_b).wait()

        @pl.when(k > 0)
        def _():
            drain_write(h0, tbuf_a)

        transpose(rows_a, tbuf_a)
        write_out(h0, tbuf_a)

        gather(jnp.minimum(h1 + 1, HIST - 1), rows_a).wait()

        @pl.when(k > 0)
        def _():
            drain_write(h1, tbuf_b)

        transpose(rows_b, tbuf_b)
        write_out(h1, tbuf_b)
        return carry

    lax.fori_loop(0, HIST // 2, step, 0)

    pltpu.make_async_copy(w_hbm.at[idx_all.at[0]], rows_a, sem_g).wait()
    drain_write(HIST - 2, tbuf_a)
    drain_write(HIST - 1, tbuf_b)


@jax.jit
def _lookup(xT, W):
    k = pl.kernel(
        _body,
        out_type=jax.ShapeDtypeStruct((HIST, 2, TJ, 8, 128), jnp.float32),
        mesh=plsc.VectorSubcoreMesh(core_axis_name="c", subcore_axis_name="s"),
        scratch_types=[
            pltpu.VMEM((HIST, BW), jnp.int32),
            pltpu.VMEM((BW, EMBED_DIM), jnp.float32),
            pltpu.VMEM((BW, EMBED_DIM), jnp.float32),
            pltpu.VMEM((2, TJ_PER_W, 8, 128), jnp.float32),
            pltpu.VMEM((2, TJ_PER_W, 8, 128), jnp.float32),
            pltpu.SemaphoreType.DMA,
            pltpu.SemaphoreType.DMA,
        ],
        compiler_params=pltpu.CompilerParams(
            use_tc_tiling_on_sc=False, needs_layout_passes=False
        ),
    )
    return k(xT, W)


def kernel(x, W):
    out5 = _lookup(x.T, W)
    return out5.transpose(2, 4, 0, 1, 3).reshape(BATCH, HIST, EMBED_DIM)


# speedup vs baseline: 1.0065x; 1.0065x over previous
import functools

import jax
import jax.numpy as jnp
from jax import lax
from jax.experimental import pallas as pl
from jax.experimental.pallas import tpu as pltpu
from jax.experimental.pallas import tpu_sc as plsc

N_ROWS = 1000000
EMBED_DIM = 16
BATCH = 16384
HIST = 50

NC = 2
NS = 16
NW = NC * NS
TJ = BATCH // 128
TJ_PER_W = TJ // NW
BW = 128 * TJ_PER_W


def _body(
    xT_hbm, w_hbm, out_hbm,
    idx_all, rows_a, rows_b, tbuf_a, tbuf_b,
    sem_g, sem_o,
):
    wid = lax.axis_index("s") * NC + lax.axis_index("c")
    base = wid * BW

    d_iota = lax.iota(jnp.int32, EMBED_DIM)
    ti_idx = d_iota // 8
    r_idx = d_iota % 8
    zeros = jnp.zeros((EMBED_DIM,), jnp.int32)

    pltpu.sync_copy(xT_hbm.at[:, pl.ds(base, BW)], idx_all)

    def gather(h, rows_v):
        return pltpu.async_copy(w_hbm.at[idx_all.at[h]], rows_v, sem_g)

    def transpose(rows_v, tbuf):
        for j in range(TJ_PER_W):
            j_splat = zeros + j

            def per_c(c, carry):
                vals = rows_v[j * 128 + c]
                plsc.store_scatter(
                    tbuf, [ti_idx, j_splat, r_idx, zeros + c], vals
                )
                return carry

            lax.fori_loop(0, 128, per_c, 0, unroll=8)

    def write_out(h, tbuf):
        pltpu.async_copy(
            tbuf, out_hbm.at[h, :, pl.ds(wid * TJ_PER_W, TJ_PER_W)], sem_o
        )

    def drain_write(h, tbuf):
        pltpu.make_async_copy(
            tbuf, out_hbm.at[h, :, pl.ds(wid * TJ_PER_W, TJ_PER_W)], sem_o
        ).wait()

    gather(0, rows_a)

    def step(k, carry):
        h0 = 2 * k
        h1 = 2 * k + 1
        gather(h1, rows_b).wait()

        @pl.when(k > 0)
        def _():
            drain_write(h0, tbuf_a)

        transpose(rows_a, tbuf_a)
        write_out(h0, tbuf_a)

        gather(jnp.minimum(h1 + 1, HIST - 1), rows_a).wait()

        @pl.when(k > 0)
        def _():
            drain_write(h1, tbuf_b)

        transpose(rows_b, tbuf_b)
        write_out(h1, tbuf_b)
        return carry

    lax.fori_loop(0, HIST // 2, step, 0)

    pltpu.make_async_copy(w_hbm.at[idx_all.at[0]], rows_a, sem_g).wait()
    drain_write(HIST - 2, tbuf_a)
    drain_write(HIST - 1, tbuf_b)


@jax.jit
def _lookup(xT, W):
    k = pl.kernel(
        _body,
        out_type=jax.ShapeDtypeStruct((HIST, 2, TJ, 8, 128), jnp.float32),
        mesh=plsc.VectorSubcoreMesh(core_axis_name="c", subcore_axis_name="s"),
        scratch_types=[
            pltpu.VMEM((HIST, BW), jnp.int32),
            pltpu.VMEM((BW, EMBED_DIM), jnp.float32),
            pltpu.VMEM((BW, EMBED_DIM), jnp.float32),
            pltpu.VMEM((2, TJ_PER_W, 8, 128), jnp.float32),
            pltpu.VMEM((2, TJ_PER_W, 8, 128), jnp.float32),
            pltpu.SemaphoreType.DMA,
            pltpu.SemaphoreType.DMA,
        ],
        compiler_params=pltpu.CompilerParams(
            use_tc_tiling_on_sc=False, needs_layout_passes=False
        ),
    )
    return k(xT, W)


def kernel(x, W):
    out5 = _lookup(x.T, W)
    return out5.transpose(2, 4, 0, 1, 3).reshape(BATCH, HIST, EMBED_DIM)


# speedup vs baseline: 1.1164x; 1.1091x over previous
import functools

import jax
import jax.numpy as jnp
from jax import lax
from jax.experimental import pallas as pl
from jax.experimental.pallas import tpu as pltpu
from jax.experimental.pallas import tpu_sc as plsc

N_ROWS = 1000000
EMBED_DIM = 16
BATCH = 16384
HIST = 50

NC = 2
NS = 16
NW = NC * NS
TJ = BATCH // 128
TJ_PER_W = TJ // NW
BW = 128 * TJ_PER_W


def _body(
    xT_hbm, w_hbm, out_hbm,
    idx_all, rows_a, rows_b, tbuf_a, tbuf_b,
    sem_g, sem_o,
):
    wid = lax.axis_index("s") * NC + lax.axis_index("c")
    base = wid * BW

    d_iota = lax.iota(jnp.int32, EMBED_DIM)
    ti_idx = d_iota // 8
    r_idx = d_iota % 8
    zeros = jnp.zeros((EMBED_DIM,), jnp.int32)

    pltpu.sync_copy(xT_hbm.at[:, pl.ds(base, BW)], idx_all)

    def gather(h, rows_v):
        return pltpu.async_copy(w_hbm.at[idx_all.at[h]], rows_v, sem_g)

    def transpose(rows_v, tbuf):
        for j in range(TJ_PER_W):
            j_splat = zeros + j

            @plsc.parallel_loop(0, 128, unroll=8)
            def per_c(c, _j=j, _js=j_splat):
                vals = rows_v[_j * 128 + c]
                plsc.store_scatter(
                    tbuf, [ti_idx, _js, r_idx, zeros + c], vals
                )

    def write_out(h, tbuf):
        pltpu.async_copy(
            tbuf, out_hbm.at[h, :, pl.ds(wid * TJ_PER_W, TJ_PER_W)], sem_o
        )

    def drain_write(h, tbuf):
        pltpu.make_async_copy(
            tbuf, out_hbm.at[h, :, pl.ds(wid * TJ_PER_W, TJ_PER_W)], sem_o
        ).wait()

    gather(0, rows_a)

    def step(k, carry):
        h0 = 2 * k
        h1 = 2 * k + 1
        gather(h1, rows_b).wait()

        @pl.when(k > 0)
        def _():
            drain_write(h0, tbuf_a)

        transpose(rows_a, tbuf_a)
        write_out(h0, tbuf_a)

        gather(jnp.minimum(h1 + 1, HIST - 1), rows_a).wait()

        @pl.when(k > 0)
        def _():
            drain_write(h1, tbuf_b)

        transpose(rows_b, tbuf_b)
        write_out(h1, tbuf_b)
        return carry

    lax.fori_loop(0, HIST // 2, step, 0)

    pltpu.make_async_copy(w_hbm.at[idx_all.at[0]], rows_a, sem_g).wait()
    drain_write(HIST - 2, tbuf_a)
    drain_write(HIST - 1, tbuf_b)


@jax.jit
def _lookup(xT, W):
    k = pl.kernel(
        _body,
        out_type=jax.ShapeDtypeStruct((HIST, 2, TJ, 8, 128), jnp.float32),
        mesh=plsc.VectorSubcoreMesh(core_axis_name="c", subcore_axis_name="s"),
        scratch_types=[
            pltpu.VMEM((HIST, BW), jnp.int32),
            pltpu.VMEM((BW, EMBED_DIM), jnp.float32),
            pltpu.VMEM((BW, EMBED_DIM), jnp.float32),
            pltpu.VMEM((2, TJ_PER_W, 8, 128), jnp.float32),
            pltpu.VMEM((2, TJ_PER_W, 8, 128), jnp.float32),
            pltpu.SemaphoreType.DMA,
            pltpu.SemaphoreType.DMA,
        ],
        compiler_params=pltpu.CompilerParams(
            use_tc_tiling_on_sc=False, needs_layout_passes=False
        ),
    )
    return k(xT, W)


def kernel(x, W):
    out5 = _lookup(x.T, W)
    return out5.transpose(2, 4, 0, 1, 3).reshape(BATCH, HIST, EMBED_DIM)


# speedup vs baseline: 1.1169x; 1.0004x over previous
import functools

import jax
import jax.numpy as jnp
from jax import lax
from jax.experimental import pallas as pl
from jax.experimental.pallas import tpu as pltpu
from jax.experimental.pallas import tpu_sc as plsc

N_ROWS = 1000000
EMBED_DIM = 16
BATCH = 16384
HIST = 50

NC = 2
NS = 16
NW = NC * NS
TJ = BATCH // 128
TJ_PER_W = TJ // NW
BW = 128 * TJ_PER_W


def _body(
    xT_hbm, w_hbm, out_hbm,
    idx_all, rows_a, rows_b, tbuf_a, tbuf_b,
    sem_g, sem_o,
):
    wid = lax.axis_index("s") * NC + lax.axis_index("c")
    base = wid * BW

    d_iota = lax.iota(jnp.int32, EMBED_DIM)
    ti_idx = d_iota // 8
    r_idx = d_iota % 8
    zeros = jnp.zeros((EMBED_DIM,), jnp.int32)

    pltpu.sync_copy(xT_hbm.at[:, pl.ds(base, BW)], idx_all)

    def gather(h, rows_v):
        return pltpu.async_copy(w_hbm.at[idx_all.at[h]], rows_v, sem_g)

    def transpose(rows_v, tbuf):
        for j in range(TJ_PER_W):
            j_splat = zeros + j

            @plsc.parallel_loop(0, 128, unroll=16)
            def per_c(c, _j=j, _js=j_splat):
                vals = rows_v[_j * 128 + c]
                plsc.store_scatter(
                    tbuf, [ti_idx, _js, r_idx, zeros + c], vals
                )

    def write_out(h, tbuf):
        pltpu.async_copy(
            tbuf, out_hbm.at[h, :, pl.ds(wid * TJ_PER_W, TJ_PER_W)], sem_o
        )

    def drain_write(h, tbuf):
        pltpu.make_async_copy(
            tbuf, out_hbm.at[h, :, pl.ds(wid * TJ_PER_W, TJ_PER_W)], sem_o
        ).wait()

    gather(0, rows_a)

    def step(k, carry):
        h0 = 2 * k
        h1 = 2 * k + 1
        gather(h1, rows_b).wait()

        @pl.when(k > 0)
        def _():
            drain_write(h0, tbuf_a)

        transpose(rows_a, tbuf_a)
        write_out(h0, tbuf_a)

        gather(jnp.minimum(h1 + 1, HIST - 1), rows_a).wait()

        @pl.when(k > 0)
        def _():
            drain_write(h1, tbuf_b)

        transpose(rows_b, tbuf_b)
        write_out(h1, tbuf_b)
        return carry

    lax.fori_loop(0, HIST // 2, step, 0)

    pltpu.make_async_copy(w_hbm.at[idx_all.at[0]], rows_a, sem_g).wait()
    drain_write(HIST - 2, tbuf_a)
    drain_write(HIST - 1, tbuf_b)


@jax.jit
def _lookup(xT, W):
    k = pl.kernel(
        _body,
        out_type=jax.ShapeDtypeStruct((HIST, 2, TJ, 8, 128), jnp.float32),
        mesh=plsc.VectorSubcoreMesh(core_axis_name="c", subcore_axis_name="s"),
        scratch_types=[
            pltpu.VMEM((HIST, BW), jnp.int32),
            pltpu.VMEM((BW, EMBED_DIM), jnp.float32),
            pltpu.VMEM((BW, EMBED_DIM), jnp.float32),
            pltpu.VMEM((2, TJ_PER_W, 8, 128), jnp.float32),
            pltpu.VMEM((2, TJ_PER_W, 8, 128), jnp.float32),
            pltpu.SemaphoreType.DMA,
            pltpu.SemaphoreType.DMA,
        ],
        compiler_params=pltpu.CompilerParams(
            use_tc_tiling_on_sc=False, needs_layout_passes=False
        ),
    )
    return k(xT, W)


def kernel(x, W):
    out5 = _lookup(x.T, W)
    return out5.transpose(2, 4, 0, 1, 3).reshape(BATCH, HIST, EMBED_DIM)
